# baseline (speedup 1.0000x reference)
import functools

import jax
import jax.numpy as jnp
from jax import lax
from jax.experimental import pallas as pl
from jax.experimental.pallas import tpu as pltpu
from jax.experimental.pallas import tpu_sc as plsc

N = 10000
E = 160000
C = 16
M = 32
G = 16
CG = C * G

NC = 2
NS = 16
HALF = N // NC
TRASH = HALF
ACC_ROWS = 5120
KC = 64
SUBS = 8
SUPER = KC * SUBS
NSUP = 20
EDGES_PER_SUB = NSUP * SUPER
E_PAD = NS * EDGES_PER_SUB


def _sc_segment_sum(ph2, dst_pad, src_pad):
  mesh = plsc.VectorSubcoreMesh(core_axis_name="c", subcore_axis_name="s")

  @functools.partial(
      pl.kernel,
      out_type=(
          jax.ShapeDtypeStruct((N, CG), jnp.float32),
          jax.ShapeDtypeStruct((N, 16), jnp.float32),
      ),
      mesh=mesh,
      compiler_params=pltpu.CompilerParams(use_tc_tiling_on_sc=False),
      scratch_types=[
          pltpu.VMEM_SHARED((ACC_ROWS, CG), jnp.float32),
          pltpu.VMEM_SHARED((ACC_ROWS, 16), jnp.float32),
          pltpu.VMEM((SUPER,), jnp.int32),
          pltpu.VMEM((SUPER,), jnp.int32),
          pltpu.VMEM((SUBS, KC), jnp.int32),
          pltpu.VMEM((SUBS, KC), jnp.int32),
          pltpu.VMEM((KC, CG), jnp.float32),
          pltpu.VMEM((KC, CG), jnp.float32),
          pltpu.VMEM((KC, 16), jnp.float32),
          pltpu.VMEM((KC, 16), jnp.float32),
          pltpu.SemaphoreType.DMA,
          pltpu.SemaphoreType.DMA,
          pltpu.SemaphoreType.DMA,
          pltpu.SemaphoreType.DMA,
          pltpu.SemaphoreType.DMA,
          pltpu.SemaphoreType.DMA,
          pltpu.SemaphoreType.DMA,
          pltpu.SemaphoreType.DMA,
      ],
  )
  def body(ph_hbm, dst_hbm, src_hbm, sums_hbm, cnts_hbm,
           acc, cacc, dst_st, src_st, ldv8, srcv8, rows0, rows1,
           ones16, z16, semid, semis, semg0, semg1, sems0, sems1,
           semc0, semc1):
    cid = lax.axis_index("c")
    sid = lax.axis_index("s")
    base = cid * HALF
    rows = [rows0, rows1]
    semg = [semg0, semg1]
    sems = [sems0, sems1]
    semc = [semc0, semc1]

    def fill_row(i, _):
      ones16[i, :] = jnp.full((16,), 1.0, jnp.float32)
      z16[i, :] = jnp.zeros((16,), jnp.float32)
      for j in range(CG // 16):
        rows0[i, pl.ds(j * 16, 16)] = jnp.zeros((16,), jnp.float32)
      return 0
    lax.fori_loop(0, KC, fill_row, 0)

    for k in range(ACC_ROWS // KC // NS):
      q = sid + NS * k
      pltpu.sync_copy(rows0, acc.at[pl.ds(q * KC, KC)])
      pltpu.sync_copy(z16, cacc.at[pl.ds(q * KC, KC)])
    plsc.subcore_barrier()

    ebase = sid * EDGES_PER_SUB

    pltpu.async_copy(dst_hbm.at[pl.ds(ebase, SUPER)], dst_st, semid)
    pltpu.async_copy(src_hbm.at[pl.ds(ebase, SUPER)], src_st, semis)

    def super_chunk(S, _):
      off = ebase + S * SUPER
      pltpu.make_async_copy(dst_hbm.at[pl.ds(off, SUPER)], dst_st,
                            semid).wait()
      pltpu.make_async_copy(src_hbm.at[pl.ds(off, SUPER)], src_st,
                            semis).wait()
      for g in range(SUPER // 16):
        d = dst_st[pl.ds(g * 16, 16)]
        sv = src_st[pl.ds(g * 16, 16)]
        l = d - base
        ok = (l >= 0) & (l < HALF)
        r, c = g // (KC // 16), (g % (KC // 16)) * 16
        ldv8[r, pl.ds(c, 16)] = jnp.where(ok, l, TRASH)
        srcv8[r, pl.ds(c, 16)] = sv
      @pl.when(S < NSUP - 1)
      def _():
        off2 = off + SUPER
        pltpu.async_copy(dst_hbm.at[pl.ds(off2, SUPER)], dst_st, semid)
        pltpu.async_copy(src_hbm.at[pl.ds(off2, SUPER)], src_st, semis)

      gd, sd, cd = {}, {}, {}
      for j in range(SUBS):
        if j >= 2:
          sd[j - 2].wait()
          cd[j - 2].wait()
        gd[j] = pltpu.async_copy(ph_hbm.at[srcv8.at[j]], rows[j % 2],
                                 semg[j % 2])
        if j >= 1:
          gd[j - 1].wait()
          sd[j - 1] = pltpu.async_copy(rows[(j - 1) % 2],
                                       acc.at[ldv8.at[j - 1]],
                                       sems[(j - 1) % 2], add=True)
          cd[j - 1] = pltpu.async_copy(ones16, cacc.at[ldv8.at[j - 1]],
                                       semc[(j - 1) % 2], add=True)
      sd[SUBS - 2].wait()
      cd[SUBS - 2].wait()
      gd[SUBS - 1].wait()
      last = SUBS - 1
      pltpu.async_copy(rows[last % 2], acc.at[ldv8.at[last]],
                       sems[last % 2], add=True).wait()
      pltpu.async_copy(ones16, cacc.at[ldv8.at[last]],
                       semc[last % 2], add=True).wait()
      return 0

    lax.fori_loop(0, NSUP, super_chunk, 0)
    plsc.subcore_barrier()

    nq = HALF // 200
    for k in range((nq + NS - 1) // NS):
      q = sid + NS * k
      @pl.when(q < nq)
      def _():
        r0 = q * 200
        pltpu.sync_copy(acc.at[pl.ds(r0, 200)],
                        sums_hbm.at[pl.ds(base + r0, 200)])
        pltpu.sync_copy(cacc.at[pl.ds(r0, 200)],
                        cnts_hbm.at[pl.ds(base + r0, 200)])

  return body(ph2, dst_pad, src_pad)


def _tc_body(sums_ref, cnts_ref, x_ref, lamqt_ref, lamb2_ref,
             logtot_ref, post_ref):
  f32 = jnp.float32
  lamqt = lamqt_ref[...]
  qm = jnp.max(lamqt, axis=1, keepdims=True)
  qe = jnp.exp(lamqt - qm)
  qs = qe / jnp.sum(qe, axis=1, keepdims=True)
  r16 = lax.broadcasted_iota(jnp.int32, (C, CG), 0)
  c256 = lax.broadcasted_iota(jnp.int32, (C, CG), 1)
  e16 = (lax.div(c256, G) == r16).astype(f32)
  qsel = jnp.dot(qs, e16, preferred_element_type=f32)
  rr = lax.broadcasted_iota(jnp.int32, (CG, CG), 0)
  cc = lax.broadcasted_iota(jnp.int32, (CG, CG), 1)
  w = qsel * (lax.rem(rr, G) == lax.rem(cc, G)).astype(f32)

  lamb2 = lamb2_ref[...]
  bm = jnp.max(lamb2, axis=0, keepdims=True)
  be = jnp.exp(lamb2 - bm)
  bs = be / jnp.sum(be, axis=0, keepdims=True)

  nb = sums_ref.shape[0]
  cm = jnp.maximum(cnts_ref[...][:, 0:1], 1.0)
  aggr = sums_ref[...] / cm

  qa = jnp.dot(aggr, w, preferred_element_type=f32)

  xb = x_ref[...]
  mio = lax.broadcasted_iota(jnp.int32, (nb, M), 1)
  oh = (xb == mio).astype(f32)
  bn = jnp.dot(oh, bs, preferred_element_type=f32)

  tmp = bn * qa
  sr = lax.broadcasted_iota(jnp.int32, (CG, G), 0)
  sc = lax.broadcasted_iota(jnp.int32, (CG, G), 1)
  s_mat = (lax.rem(sr, G) == sc).astype(f32)
  total = jnp.dot(tmp, s_mat, preferred_element_type=f32)
  tr = lax.broadcasted_iota(jnp.int32, (G, CG), 0)
  tc = lax.broadcasted_iota(jnp.int32, (G, CG), 1)
  st_mat = (tr == lax.rem(tc, G)).astype(f32)
  totb = jnp.dot(total, st_mat, preferred_element_type=f32)

  logtot_ref[...] = jnp.log(total)
  post_ref[...] = tmp / totb


def _tc_dense(sums, cnts, x2, lamqt, lamb2):
  nb = 1000
  grid = N // nb
  return pl.pallas_call(
      _tc_body,
      grid=(grid,),
      in_specs=[
          pl.BlockSpec((nb, CG), lambda i: (i, 0)),
          pl.BlockSpec((nb, 16), lambda i: (i, 0)),
          pl.BlockSpec((nb, 1), lambda i: (i, 0)),
          pl.BlockSpec((CG, C), lambda i: (0, 0)),
          pl.BlockSpec((M, CG), lambda i: (0, 0)),
      ],
      out_specs=[
          pl.BlockSpec((nb, G), lambda i: (i, 0)),
          pl.BlockSpec((nb, CG), lambda i: (i, 0)),
      ],
      out_shape=[
          jax.ShapeDtypeStruct((N, G), jnp.float32),
          jax.ShapeDtypeStruct((N, CG), jnp.float32),
      ],
  )(sums, cnts, x2, lamqt, lamb2)


def kernel(x, prev_h, edge_index, lambda_Q, lambda_B):
  ph2 = prev_h.reshape(N, CG)
  dst = edge_index[0]
  src = edge_index[1]
  pad = E_PAD - E
  dst_pad = jnp.concatenate([dst, jnp.full((pad,), -1, dst.dtype)])
  src_pad = jnp.concatenate([src, jnp.zeros((pad,), src.dtype)])

  sums, cnts = _sc_segment_sum(ph2, dst_pad.astype(jnp.int32),
                               src_pad.astype(jnp.int32))

  lamqt = jnp.transpose(lambda_Q, (1, 2, 0)).reshape(CG, C)
  lamb2 = jnp.transpose(lambda_B, (1, 0, 2)).reshape(M, CG)
  x2 = x.reshape(N, 1).astype(jnp.int32)

  logtot, post = _tc_dense(sums, cnts, x2, lamqt, lamb2)
  return (logtot, post.reshape(N, C, G))


# speedup vs baseline: 1.8282x; 1.8282x over previous
import functools

import jax
import jax.numpy as jnp
from jax import lax
from jax.experimental import pallas as pl
from jax.experimental.pallas import tpu as pltpu
from jax.experimental.pallas import tpu_sc as plsc

N = 10000
E = 160000
C = 16
M = 32
G = 16
CG = C * G

NC = 2
NS = 16
HALF = N // NC
TRASH = HALF
ACC_ROWS = 5120
KC = 128
SUPER = 512
NSUP = 20
EDGES_PER_SUB = NSUP * SUPER
E_PAD = NS * EDGES_PER_SUB
CAP = 768


def _sc_segment_sum(ph2, dst_pad, src_pad):
  mesh = plsc.VectorSubcoreMesh(core_axis_name="c", subcore_axis_name="s")

  @functools.partial(
      pl.kernel,
      out_type=(
          jax.ShapeDtypeStruct((N, CG), jnp.float32),
          jax.ShapeDtypeStruct((N, 16), jnp.float32),
      ),
      mesh=mesh,
      compiler_params=pltpu.CompilerParams(use_tc_tiling_on_sc=False,
                                           needs_layout_passes=False),
      scratch_types=[
          pltpu.VMEM_SHARED((ACC_ROWS, CG), jnp.float32),
          pltpu.VMEM_SHARED((ACC_ROWS, 16), jnp.float32),
          pltpu.VMEM((SUPER,), jnp.int32),
          pltpu.VMEM((SUPER,), jnp.int32),
          pltpu.VMEM((CAP,), jnp.int32),
          pltpu.VMEM((CAP,), jnp.int32),
          pltpu.VMEM((1, KC), jnp.int32),
          pltpu.VMEM((1, KC), jnp.int32),
          pltpu.VMEM((KC, CG), jnp.float32),
          pltpu.VMEM((KC, 16), jnp.float32),
          pltpu.VMEM((KC, 16), jnp.float32),
          pltpu.SemaphoreType.DMA,
          pltpu.SemaphoreType.DMA,
          pltpu.SemaphoreType.DMA,
      ],
  )
  def body(ph_hbm, dst_hbm, src_hbm, sums_hbm, cnts_hbm,
           acc, cacc, dst_st, src_st, ldc, srcc, ldx, srcx, rows,
           ones16, z16, semid, semis, semg):
    cid = lax.axis_index("c")
    sid = lax.axis_index("s")
    base = cid * HALF

    def fill_row(i, _):
      ones16[i, :] = jnp.full((16,), 1.0, jnp.float32)
      z16[i, :] = jnp.zeros((16,), jnp.float32)
      for j in range(CG // 16):
        rows[i, pl.ds(j * 16, 16)] = jnp.zeros((16,), jnp.float32)
      return 0
    lax.fori_loop(0, KC, fill_row, 0)

    for k in range(ACC_ROWS // KC // NS):
      q = sid + NS * k
      pltpu.sync_copy(rows, acc.at[pl.ds(q * KC, KC)])
      pltpu.sync_copy(z16, cacc.at[pl.ds(q * KC, KC)])
    plsc.subcore_barrier()

    ebase = sid * EDGES_PER_SUB

    def process_chunk(t):
      for g in range(KC // 16):
        ldx[0, pl.ds(g * 16, 16)] = ldc[pl.ds(t * KC + g * 16, 16)]
        srcx[0, pl.ds(g * 16, 16)] = srcc[pl.ds(t * KC + g * 16, 16)]
      pltpu.async_copy(ph_hbm.at[srcx.at[0]], rows, semg).wait()
      pltpu.sync_copy(rows, acc.at[ldx.at[0]], add=True)
      pltpu.sync_copy(ones16, cacc.at[ldx.at[0]], add=True)

    pltpu.async_copy(dst_hbm.at[pl.ds(ebase, SUPER)], dst_st, semid)
    pltpu.async_copy(src_hbm.at[pl.ds(ebase, SUPER)], src_st, semis)

    def super_chunk(S, n):
      off = ebase + S * SUPER
      pltpu.make_async_copy(dst_hbm.at[pl.ds(off, SUPER)], dst_st,
                            semid).wait()
      pltpu.make_async_copy(src_hbm.at[pl.ds(off, SUPER)], src_st,
                            semis).wait()
      for g in range(SUPER // 16):
        d = dst_st[pl.ds(g * 16, 16)]
        sv = src_st[pl.ds(g * 16, 16)]
        l = d - base
        ok = (l >= 0) & (l < HALF)
        oki = jnp.where(ok, jnp.full((16,), 1, jnp.int32),
                        jnp.zeros((16,), jnp.int32))
        incl = plsc.cumsum(oki)
        lane = lax.iota(jnp.int32, 16)
        nv = jnp.full((16,), n, jnp.int32)
        pos = jnp.where(ok, nv + incl - 1, CAP - 16 + lane)
        plsc.store_scatter(ldc, [pos], l)
        plsc.store_scatter(srcc, [pos], sv)
        n = n + jnp.max(incl)
      @pl.when(S < NSUP - 1)
      def _():
        off2 = off + SUPER
        pltpu.async_copy(dst_hbm.at[pl.ds(off2, SUPER)], dst_st, semid)
        pltpu.async_copy(src_hbm.at[pl.ds(off2, SUPER)], src_st, semis)

      nchunks = n // KC
      lax.fori_loop(0, nchunks, lambda t, _: (process_chunk(t), 0)[1], 0)
      rem_off = nchunks * KC
      for g in range(KC // 16):
        lv = ldc[pl.ds(rem_off + g * 16, 16)]
        sv = srcc[pl.ds(rem_off + g * 16, 16)]
        ldc[pl.ds(g * 16, 16)] = lv
        srcc[pl.ds(g * 16, 16)] = sv
      return n - rem_off

    n_fin = lax.fori_loop(0, NSUP, super_chunk, jnp.int32(0))

    @pl.when(n_fin > 0)
    def _():
      for g in range(KC // 16):
        ldc[pl.ds(n_fin + g * 16, 16)] = jnp.full((16,), TRASH, jnp.int32)
        srcc[pl.ds(n_fin + g * 16, 16)] = jnp.zeros((16,), jnp.int32)
      process_chunk(0)

    plsc.subcore_barrier()

    nq = HALF // 200
    for k in range((nq + NS - 1) // NS):
      q = sid + NS * k
      @pl.when(q < nq)
      def _():
        r0 = q * 200
        pltpu.sync_copy(acc.at[pl.ds(r0, 200)],
                        sums_hbm.at[pl.ds(base + r0, 200)])
        pltpu.sync_copy(cacc.at[pl.ds(r0, 200)],
                        cnts_hbm.at[pl.ds(base + r0, 200)])

  return body(ph2, dst_pad, src_pad)


def _tc_body(sums_ref, cnts_ref, x_ref, lamqt_ref, lamb2_ref,
             logtot_ref, post_ref):
  f32 = jnp.float32
  lamqt = lamqt_ref[...]
  qm = jnp.max(lamqt, axis=1, keepdims=True)
  qe = jnp.exp(lamqt - qm)
  qs = qe / jnp.sum(qe, axis=1, keepdims=True)
  r16 = lax.broadcasted_iota(jnp.int32, (C, CG), 0)
  c256 = lax.broadcasted_iota(jnp.int32, (C, CG), 1)
  e16 = (lax.div(c256, G) == r16).astype(f32)
  qsel = jnp.dot(qs, e16, preferred_element_type=f32)
  rr = lax.broadcasted_iota(jnp.int32, (CG, CG), 0)
  cc = lax.broadcasted_iota(jnp.int32, (CG, CG), 1)
  w = qsel * (lax.rem(rr, G) == lax.rem(cc, G)).astype(f32)

  lamb2 = lamb2_ref[...]
  bm = jnp.max(lamb2, axis=0, keepdims=True)
  be = jnp.exp(lamb2 - bm)
  bs = be / jnp.sum(be, axis=0, keepdims=True)

  nb = sums_ref.shape[0]
  cm = jnp.maximum(cnts_ref[...][:, 0:1], 1.0)
  aggr = sums_ref[...] / cm

  qa = jnp.dot(aggr, w, preferred_element_type=f32)

  xb = x_ref[...]
  mio = lax.broadcasted_iota(jnp.int32, (nb, M), 1)
  oh = (xb == mio).astype(f32)
  bn = jnp.dot(oh, bs, preferred_element_type=f32)

  tmp = bn * qa
  sr = lax.broadcasted_iota(jnp.int32, (CG, G), 0)
  sc = lax.broadcasted_iota(jnp.int32, (CG, G), 1)
  s_mat = (lax.rem(sr, G) == sc).astype(f32)
  total = jnp.dot(tmp, s_mat, preferred_element_type=f32)
  tr = lax.broadcasted_iota(jnp.int32, (G, CG), 0)
  tc = lax.broadcasted_iota(jnp.int32, (G, CG), 1)
  st_mat = (tr == lax.rem(tc, G)).astype(f32)
  totb = jnp.dot(total, st_mat, preferred_element_type=f32)

  logtot_ref[...] = jnp.log(total)
  post_ref[...] = tmp / totb


def _tc_dense(sums, cnts, x2, lamqt, lamb2):
  nb = 1000
  grid = N // nb
  return pl.pallas_call(
      _tc_body,
      grid=(grid,),
      in_specs=[
          pl.BlockSpec((nb, CG), lambda i: (i, 0)),
          pl.BlockSpec((nb, 16), lambda i: (i, 0)),
          pl.BlockSpec((nb, 1), lambda i: (i, 0)),
          pl.BlockSpec((CG, C), lambda i: (0, 0)),
          pl.BlockSpec((M, CG), lambda i: (0, 0)),
      ],
      out_specs=[
          pl.BlockSpec((nb, G), lambda i: (i, 0)),
          pl.BlockSpec((nb, CG), lambda i: (i, 0)),
      ],
      out_shape=[
          jax.ShapeDtypeStruct((N, G), jnp.float32),
          jax.ShapeDtypeStruct((N, CG), jnp.float32),
      ],
  )(sums, cnts, x2, lamqt, lamb2)


def kernel(x, prev_h, edge_index, lambda_Q, lambda_B):
  ph2 = prev_h.reshape(N, CG)
  dst = edge_index[0]
  src = edge_index[1]
  pad = E_PAD - E
  dst_pad = jnp.concatenate([dst, jnp.full((pad,), -1, dst.dtype)])
  src_pad = jnp.concatenate([src, jnp.zeros((pad,), src.dtype)])

  sums, cnts = _sc_segment_sum(ph2, dst_pad.astype(jnp.int32),
                               src_pad.astype(jnp.int32))

  lamqt = jnp.transpose(lambda_Q, (1, 2, 0)).reshape(CG, C)
  lamb2 = jnp.transpose(lambda_B, (1, 0, 2)).reshape(M, CG)
  x2 = x.reshape(N, 1).astype(jnp.int32)

  logtot, post = _tc_dense(sums, cnts, x2, lamqt, lamb2)
  return (logtot, post.reshape(N, C, G))


# speedup vs baseline: 1.8695x; 1.0226x over previous
import functools

import jax
import jax.numpy as jnp
from jax import lax
from jax.experimental import pallas as pl
from jax.experimental.pallas import tpu as pltpu
from jax.experimental.pallas import tpu_sc as plsc

N = 10000
E = 160000
C = 16
M = 32
G = 16
CG = C * G

NC = 2
NS = 16
HALF = N // NC
TRASH = HALF
ACC_ROWS = 5120
KC = 128
SUPER = 512
NSUP = 20
EDGES_PER_SUB = NSUP * SUPER
E_PAD = NS * EDGES_PER_SUB
CAP = 768


def _sc_segment_sum(ph2, dst_pad, src_pad):
  mesh = plsc.VectorSubcoreMesh(core_axis_name="c", subcore_axis_name="s")

  @functools.partial(
      pl.kernel,
      out_type=(
          jax.ShapeDtypeStruct((N, CG), jnp.float32),
          jax.ShapeDtypeStruct((N, 16), jnp.float32),
      ),
      mesh=mesh,
      compiler_params=pltpu.CompilerParams(use_tc_tiling_on_sc=False,
                                           needs_layout_passes=False),
      scratch_types=[
          pltpu.VMEM_SHARED((ACC_ROWS, CG), jnp.float32),
          pltpu.VMEM_SHARED((ACC_ROWS, 16), jnp.float32),
          pltpu.VMEM((SUPER,), jnp.int32),
          pltpu.VMEM((SUPER,), jnp.int32),
          pltpu.VMEM((CAP,), jnp.int32),
          pltpu.VMEM((CAP,), jnp.int32),
          pltpu.VMEM((1, KC), jnp.int32),
          pltpu.VMEM((KC, CG), jnp.float32),
          pltpu.VMEM((KC, 16), jnp.float32),
          pltpu.VMEM((KC, 16), jnp.float32),
          pltpu.SemaphoreType.DMA,
          pltpu.SemaphoreType.DMA,
          pltpu.SemaphoreType.DMA,
          pltpu.SemaphoreType.DMA,
      ],
  )
  def body(ph_hbm, dst_hbm, src_hbm, sums_hbm, cnts_hbm,
           acc, cacc, dst_st, src_st, ldc, srcc, ldx, rows,
           ones16, z16, semid, semis, semg, semc):
    cid = lax.axis_index("c")
    sid = lax.axis_index("s")
    base = cid * HALF

    def fill_row(i, _):
      ones16[i, :] = jnp.full((16,), 1.0, jnp.float32)
      z16[i, :] = jnp.zeros((16,), jnp.float32)
      for j in range(CG // 16):
        rows[i, pl.ds(j * 16, 16)] = jnp.zeros((16,), jnp.float32)
      return 0
    lax.fori_loop(0, KC, fill_row, 0)

    for k in range(ACC_ROWS // KC // NS):
      q = sid + NS * k
      pltpu.sync_copy(rows, acc.at[pl.ds(q * KC, KC)])
      pltpu.sync_copy(z16, cacc.at[pl.ds(q * KC, KC)])
    plsc.subcore_barrier()

    ebase = sid * EDGES_PER_SUB

    def process_chunk(t):
      gd = pltpu.async_copy(ph_hbm.at[srcc.at[pl.ds(t * KC, KC)]], rows,
                            semg)
      for g in range(KC // 16):
        ldx[0, pl.ds(g * 16, 16)] = ldc[pl.ds(t * KC + g * 16, 16)]
      cd = pltpu.async_copy(ones16, cacc.at[ldx.at[0]], semc, add=True)
      gd.wait()
      pltpu.sync_copy(rows, acc.at[ldx.at[0]], add=True)
      cd.wait()

    pltpu.async_copy(dst_hbm.at[pl.ds(ebase, SUPER)], dst_st, semid)
    pltpu.async_copy(src_hbm.at[pl.ds(ebase, SUPER)], src_st, semis)

    def super_chunk(S, n):
      off = ebase + S * SUPER
      pltpu.make_async_copy(dst_hbm.at[pl.ds(off, SUPER)], dst_st,
                            semid).wait()
      pltpu.make_async_copy(src_hbm.at[pl.ds(off, SUPER)], src_st,
                            semis).wait()
      for g in range(SUPER // 16):
        d = dst_st[pl.ds(g * 16, 16)]
        sv = src_st[pl.ds(g * 16, 16)]
        l = d - base
        ok = (l >= 0) & (l < HALF)
        oki = jnp.where(ok, jnp.full((16,), 1, jnp.int32),
                        jnp.zeros((16,), jnp.int32))
        incl = plsc.cumsum(oki)
        lane = lax.iota(jnp.int32, 16)
        nv = jnp.full((16,), n, jnp.int32)
        pos = jnp.where(ok, nv + incl - 1, CAP - 16 + lane)
        plsc.store_scatter(ldc, [pos], l)
        plsc.store_scatter(srcc, [pos], sv)
        n = n + jnp.max(incl)
      @pl.when(S < NSUP - 1)
      def _():
        off2 = off + SUPER
        pltpu.async_copy(dst_hbm.at[pl.ds(off2, SUPER)], dst_st, semid)
        pltpu.async_copy(src_hbm.at[pl.ds(off2, SUPER)], src_st, semis)

      nchunks = n // KC
      lax.fori_loop(0, nchunks, lambda t, _: (process_chunk(t), 0)[1], 0)
      rem_off = nchunks * KC
      for g in range(KC // 16):
        lv = ldc[pl.ds(rem_off + g * 16, 16)]
        sv = srcc[pl.ds(rem_off + g * 16, 16)]
        ldc[pl.ds(g * 16, 16)] = lv
        srcc[pl.ds(g * 16, 16)] = sv
      return n - rem_off

    n_fin = lax.fori_loop(0, NSUP, super_chunk, jnp.int32(0))

    @pl.when(n_fin > 0)
    def _():
      for g in range(KC // 16):
        ldc[pl.ds(n_fin + g * 16, 16)] = jnp.full((16,), TRASH, jnp.int32)
        srcc[pl.ds(n_fin + g * 16, 16)] = jnp.zeros((16,), jnp.int32)
      process_chunk(0)

    plsc.subcore_barrier()

    nq = HALF // 200
    for k in range((nq + NS - 1) // NS):
      q = sid + NS * k
      @pl.when(q < nq)
      def _():
        r0 = q * 200
        pltpu.sync_copy(acc.at[pl.ds(r0, 200)],
                        sums_hbm.at[pl.ds(base + r0, 200)])
        pltpu.sync_copy(cacc.at[pl.ds(r0, 200)],
                        cnts_hbm.at[pl.ds(base + r0, 200)])

  return body(ph2, dst_pad, src_pad)


def _tc_body(sums_ref, cnts_ref, x_ref, lamqt_ref, lamb2_ref,
             logtot_ref, post_ref):
  f32 = jnp.float32
  lamqt = lamqt_ref[...]
  qm = jnp.max(lamqt, axis=1, keepdims=True)
  qe = jnp.exp(lamqt - qm)
  qs = qe / jnp.sum(qe, axis=1, keepdims=True)
  r16 = lax.broadcasted_iota(jnp.int32, (C, CG), 0)
  c256 = lax.broadcasted_iota(jnp.int32, (C, CG), 1)
  e16 = (lax.div(c256, G) == r16).astype(f32)
  qsel = jnp.dot(qs, e16, preferred_element_type=f32)
  rr = lax.broadcasted_iota(jnp.int32, (CG, CG), 0)
  cc = lax.broadcasted_iota(jnp.int32, (CG, CG), 1)
  w = qsel * (lax.rem(rr, G) == lax.rem(cc, G)).astype(f32)

  lamb2 = lamb2_ref[...]
  bm = jnp.max(lamb2, axis=0, keepdims=True)
  be = jnp.exp(lamb2 - bm)
  bs = be / jnp.sum(be, axis=0, keepdims=True)

  nb = sums_ref.shape[0]
  cm = jnp.maximum(cnts_ref[...][:, 0:1], 1.0)
  aggr = sums_ref[...] / cm

  qa = jnp.dot(aggr, w, preferred_element_type=f32)

  xb = x_ref[...]
  mio = lax.broadcasted_iota(jnp.int32, (nb, M), 1)
  oh = (xb == mio).astype(f32)
  bn = jnp.dot(oh, bs, preferred_element_type=f32)

  tmp = bn * qa
  sr = lax.broadcasted_iota(jnp.int32, (CG, G), 0)
  sc = lax.broadcasted_iota(jnp.int32, (CG, G), 1)
  s_mat = (lax.rem(sr, G) == sc).astype(f32)
  total = jnp.dot(tmp, s_mat, preferred_element_type=f32)
  tr = lax.broadcasted_iota(jnp.int32, (G, CG), 0)
  tc = lax.broadcasted_iota(jnp.int32, (G, CG), 1)
  st_mat = (tr == lax.rem(tc, G)).astype(f32)
  totb = jnp.dot(total, st_mat, preferred_element_type=f32)

  logtot_ref[...] = jnp.log(total)
  post_ref[...] = tmp / totb


def _tc_dense(sums, cnts, x2, lamqt, lamb2):
  nb = 1000
  grid = N // nb
  return pl.pallas_call(
      _tc_body,
      grid=(grid,),
      in_specs=[
          pl.BlockSpec((nb, CG), lambda i: (i, 0)),
          pl.BlockSpec((nb, 16), lambda i: (i, 0)),
          pl.BlockSpec((nb, 1), lambda i: (i, 0)),
          pl.BlockSpec((CG, C), lambda i: (0, 0)),
          pl.BlockSpec((M, CG), lambda i: (0, 0)),
      ],
      out_specs=[
          pl.BlockSpec((nb, G), lambda i: (i, 0)),
          pl.BlockSpec((nb, CG), lambda i: (i, 0)),
      ],
      out_shape=[
          jax.ShapeDtypeStruct((N, G), jnp.float32),
          jax.ShapeDtypeStruct((N, CG), jnp.float32),
      ],
  )(sums, cnts, x2, lamqt, lamb2)


def kernel(x, prev_h, edge_index, lambda_Q, lambda_B):
  ph2 = prev_h.reshape(N, CG)
  dst = edge_index[0]
  src = edge_index[1]
  pad = E_PAD - E
  dst_pad = jnp.concatenate([dst, jnp.full((pad,), -1, dst.dtype)])
  src_pad = jnp.concatenate([src, jnp.zeros((pad,), src.dtype)])

  sums, cnts = _sc_segment_sum(ph2, dst_pad.astype(jnp.int32),
                               src_pad.astype(jnp.int32))

  lamqt = jnp.transpose(lambda_Q, (1, 2, 0)).reshape(CG, C)
  lamb2 = jnp.transpose(lambda_B, (1, 0, 2)).reshape(M, CG)
  x2 = x.reshape(N, 1).astype(jnp.int32)

  logtot, post = _tc_dense(sums, cnts, x2, lamqt, lamb2)
  return (logtot, post.reshape(N, C, G))


# speedup vs baseline: 2.0840x; 1.1147x over previous
import functools

import jax
import jax.numpy as jnp
from jax import lax
from jax.experimental import pallas as pl
from jax.experimental.pallas import tpu as pltpu
from jax.experimental.pallas import tpu_sc as plsc

N = 10000
E = 160000
C = 16
M = 32
G = 16
CG = C * G

NC = 2
NS = 16
HALF = N // NC
TRASH = HALF
ACC_ROWS = 5120
KC = 64
SUPER = 512
NSUP = 20
EDGES_PER_SUB = NSUP * SUPER
E_PAD = NS * EDGES_PER_SUB
CAP = 768


def _sc_segment_sum(ph2, dst_pad, src_pad):
  mesh = plsc.VectorSubcoreMesh(core_axis_name="c", subcore_axis_name="s")

  @functools.partial(
      pl.kernel,
      out_type=(
          jax.ShapeDtypeStruct((N, CG), jnp.float32),
          jax.ShapeDtypeStruct((N, 16), jnp.float32),
      ),
      mesh=mesh,
      compiler_params=pltpu.CompilerParams(use_tc_tiling_on_sc=False,
                                           needs_layout_passes=False),
      scratch_types=[
          pltpu.VMEM_SHARED((ACC_ROWS, CG), jnp.float32),
          pltpu.VMEM_SHARED((ACC_ROWS, 16), jnp.float32),
          pltpu.VMEM((SUPER,), jnp.int32),
          pltpu.VMEM((SUPER,), jnp.int32),
          pltpu.VMEM((CAP,), jnp.int32),
          pltpu.VMEM((CAP,), jnp.int32),
          pltpu.VMEM((1, KC), jnp.int32),
          pltpu.VMEM((1, KC), jnp.int32),
          pltpu.VMEM((KC, CG), jnp.float32),
          pltpu.VMEM((KC, CG), jnp.float32),
          pltpu.VMEM((KC, 16), jnp.float32),
          pltpu.VMEM((KC, 16), jnp.float32),
          pltpu.SemaphoreType.DMA,
          pltpu.SemaphoreType.DMA,
          pltpu.SemaphoreType.DMA,
          pltpu.SemaphoreType.DMA,
          pltpu.SemaphoreType.DMA,
          pltpu.SemaphoreType.DMA,
          pltpu.SemaphoreType.DMA,
          pltpu.SemaphoreType.DMA,
      ],
  )
  def body(ph_hbm, dst_hbm, src_hbm, sums_hbm, cnts_hbm,
           acc, cacc, dst_st, src_st, ldc, srcc, ldxa, ldxb, rowsa,
           rowsb, ones16, z16, semid, semis, semga, semgb, semsa,
           semsb, semca, semcb):
    cid = lax.axis_index("c")
    sid = lax.axis_index("s")
    base = cid * HALF

    def fill_row(i, _):
      ones16[i, :] = jnp.full((16,), 1.0, jnp.float32)
      z16[i, :] = jnp.zeros((16,), jnp.float32)
      for j in range(CG // 16):
        rowsa[i, pl.ds(j * 16, 16)] = jnp.zeros((16,), jnp.float32)
      return 0
    lax.fori_loop(0, KC, fill_row, 0)

    for k in range(ACC_ROWS // KC // NS):
      q = sid + NS * k
      pltpu.sync_copy(rowsa, acc.at[pl.ds(q * KC, KC)])
      pltpu.sync_copy(z16, cacc.at[pl.ds(q * KC, KC)])
    plsc.subcore_barrier()

    ebase = sid * EDGES_PER_SUB

    def fill_ldx(ldx, t):
      for g in range(KC // 16):
        ldx[0, pl.ds(g * 16, 16)] = ldc[pl.ds(t * KC + g * 16, 16)]

    def process_chunk(t):
      gd = pltpu.async_copy(ph_hbm.at[srcc.at[pl.ds(t * KC, KC)]], rowsa,
                            semga)
      fill_ldx(ldxa, t)
      cd = pltpu.async_copy(ones16, cacc.at[ldxa.at[0]], semca, add=True)
      gd.wait()
      pltpu.sync_copy(rowsa, acc.at[ldxa.at[0]], add=True)
      cd.wait()

    def process_pair(t2, nchunks):
      t0 = 2 * t2
      has_b = (t0 + 1) < nchunks
      ga = pltpu.async_copy(ph_hbm.at[srcc.at[pl.ds(t0 * KC, KC)]],
                            rowsa, semga)
      fill_ldx(ldxa, t0)
      ca = pltpu.async_copy(ones16, cacc.at[ldxa.at[0]], semca, add=True)

      @pl.when(has_b)
      def _():
        pltpu.async_copy(ph_hbm.at[srcc.at[pl.ds((t0 + 1) * KC, KC)]],
                         rowsb, semgb)
        fill_ldx(ldxb, t0 + 1)
        pltpu.async_copy(ones16, cacc.at[ldxb.at[0]], semcb, add=True)

      ga.wait()
      sa = pltpu.async_copy(rowsa, acc.at[ldxa.at[0]], semsa, add=True)

      @pl.when(has_b)
      def _():
        pltpu.make_async_copy(ph_hbm.at[srcc.at[pl.ds((t0 + 1) * KC, KC)]],
                              rowsb, semgb).wait()
        pltpu.async_copy(rowsb, acc.at[ldxb.at[0]], semsb, add=True)

      sa.wait()
      ca.wait()

      @pl.when(has_b)
      def _():
        pltpu.make_async_copy(rowsb, acc.at[ldxb.at[0]], semsb).wait()
        pltpu.make_async_copy(ones16, cacc.at[ldxb.at[0]], semcb).wait()

    pltpu.async_copy(dst_hbm.at[pl.ds(ebase, SUPER)], dst_st, semid)
    pltpu.async_copy(src_hbm.at[pl.ds(ebase, SUPER)], src_st, semis)

    def super_chunk(S, n):
      off = ebase + S * SUPER
      pltpu.make_async_copy(dst_hbm.at[pl.ds(off, SUPER)], dst_st,
                            semid).wait()
      pltpu.make_async_copy(src_hbm.at[pl.ds(off, SUPER)], src_st,
                            semis).wait()
      for g in range(SUPER // 16):
        d = dst_st[pl.ds(g * 16, 16)]
        sv = src_st[pl.ds(g * 16, 16)]
        l = d - base
        ok = (l >= 0) & (l < HALF)
        oki = jnp.where(ok, jnp.full((16,), 1, jnp.int32),
                        jnp.zeros((16,), jnp.int32))
        incl = plsc.cumsum(oki)
        lane = lax.iota(jnp.int32, 16)
        nv = jnp.full((16,), n, jnp.int32)
        pos = jnp.where(ok, nv + incl - 1, CAP - 16 + lane)
        plsc.store_scatter(ldc, [pos], l)
        plsc.store_scatter(srcc, [pos], sv)
        n = n + jnp.max(incl)
      @pl.when(S < NSUP - 1)
      def _():
        off2 = off + SUPER
        pltpu.async_copy(dst_hbm.at[pl.ds(off2, SUPER)], dst_st, semid)
        pltpu.async_copy(src_hbm.at[pl.ds(off2, SUPER)], src_st, semis)

      nchunks = n // KC
      npairs = (nchunks + 1) // 2
      lax.fori_loop(0, npairs,
                    lambda t2, _: (process_pair(t2, nchunks), 0)[1], 0)
      rem_off = nchunks * KC
      for g in range(KC // 16):
        lv = ldc[pl.ds(rem_off + g * 16, 16)]
        sv = srcc[pl.ds(rem_off + g * 16, 16)]
        ldc[pl.ds(g * 16, 16)] = lv
        srcc[pl.ds(g * 16, 16)] = sv
      return n - rem_off

    n_fin = lax.fori_loop(0, NSUP, super_chunk, jnp.int32(0))

    @pl.when(n_fin > 0)
    def _():
      for g in range(KC // 16):
        ldc[pl.ds(n_fin + g * 16, 16)] = jnp.full((16,), TRASH, jnp.int32)
        srcc[pl.ds(n_fin + g * 16, 16)] = jnp.zeros((16,), jnp.int32)
      process_chunk(0)

    plsc.subcore_barrier()

    nq = HALF // 200
    for k in range((nq + NS - 1) // NS):
      q = sid + NS * k
      @pl.when(q < nq)
      def _():
        r0 = q * 200
        pltpu.sync_copy(acc.at[pl.ds(r0, 200)],
                        sums_hbm.at[pl.ds(base + r0, 200)])
        pltpu.sync_copy(cacc.at[pl.ds(r0, 200)],
                        cnts_hbm.at[pl.ds(base + r0, 200)])

  return body(ph2, dst_pad, src_pad)


def _tc_body(sums_ref, cnts_ref, x_ref, lamqt_ref, lamb2_ref,
             logtot_ref, post_ref):
  f32 = jnp.float32
  lamqt = lamqt_ref[...]
  qm = jnp.max(lamqt, axis=1, keepdims=True)
  qe = jnp.exp(lamqt - qm)
  qs = qe / jnp.sum(qe, axis=1, keepdims=True)
  r16 = lax.broadcasted_iota(jnp.int32, (C, CG), 0)
  c256 = lax.broadcasted_iota(jnp.int32, (C, CG), 1)
  e16 = (lax.div(c256, G) == r16).astype(f32)
  qsel = jnp.dot(qs, e16, preferred_element_type=f32)
  rr = lax.broadcasted_iota(jnp.int32, (CG, CG), 0)
  cc = lax.broadcasted_iota(jnp.int32, (CG, CG), 1)
  w = qsel * (lax.rem(rr, G) == lax.rem(cc, G)).astype(f32)

  lamb2 = lamb2_ref[...]
  bm = jnp.max(lamb2, axis=0, keepdims=True)
  be = jnp.exp(lamb2 - bm)
  bs = be / jnp.sum(be, axis=0, keepdims=True)

  nb = sums_ref.shape[0]
  cm = jnp.maximum(cnts_ref[...][:, 0:1], 1.0)
  aggr = sums_ref[...] / cm

  qa = jnp.dot(aggr, w, preferred_element_type=f32)

  xb = x_ref[...]
  mio = lax.broadcasted_iota(jnp.int32, (nb, M), 1)
  oh = (xb == mio).astype(f32)
  bn = jnp.dot(oh, bs, preferred_element_type=f32)

  tmp = bn * qa
  sr = lax.broadcasted_iota(jnp.int32, (CG, G), 0)
  sc = lax.broadcasted_iota(jnp.int32, (CG, G), 1)
  s_mat = (lax.rem(sr, G) == sc).astype(f32)
  total = jnp.dot(tmp, s_mat, preferred_element_type=f32)
  tr = lax.broadcasted_iota(jnp.int32, (G, CG), 0)
  tc = lax.broadcasted_iota(jnp.int32, (G, CG), 1)
  st_mat = (tr == lax.rem(tc, G)).astype(f32)
  totb = jnp.dot(total, st_mat, preferred_element_type=f32)

  logtot_ref[...] = jnp.log(total)
  post_ref[...] = tmp / totb


def _tc_dense(sums, cnts, x2, lamqt, lamb2):
  nb = 1000
  grid = N // nb
  return pl.pallas_call(
      _tc_body,
      grid=(grid,),
      in_specs=[
          pl.BlockSpec((nb, CG), lambda i: (i, 0)),
          pl.BlockSpec((nb, 16), lambda i: (i, 0)),
          pl.BlockSpec((nb, 1), lambda i: (i, 0)),
          pl.BlockSpec((CG, C), lambda i: (0, 0)),
          pl.BlockSpec((M, CG), lambda i: (0, 0)),
      ],
      out_specs=[
          pl.BlockSpec((nb, G), lambda i: (i, 0)),
          pl.BlockSpec((nb, CG), lambda i: (i, 0)),
      ],
      out_shape=[
          jax.ShapeDtypeStruct((N, G), jnp.float32),
          jax.ShapeDtypeStruct((N, CG), jnp.float32),
      ],
  )(sums, cnts, x2, lamqt, lamb2)


def kernel(x, prev_h, edge_index, lambda_Q, lambda_B):
  ph2 = prev_h.reshape(N, CG)
  dst = edge_index[0]
  src = edge_index[1]
  pad = E_PAD - E
  dst_pad = jnp.concatenate([dst, jnp.full((pad,), -1, dst.dtype)])
  src_pad = jnp.concatenate([src, jnp.zeros((pad,), src.dtype)])

  sums, cnts = _sc_segment_sum(ph2, dst_pad.astype(jnp.int32),
                               src_pad.astype(jnp.int32))

  lamqt = jnp.transpose(lambda_Q, (1, 2, 0)).reshape(CG, C)
  lamb2 = jnp.transpose(lambda_B, (1, 0, 2)).reshape(M, CG)
  x2 = x.reshape(N, 1).astype(jnp.int32)

  logtot, post = _tc_dense(sums, cnts, x2, lamqt, lamb2)
  return (logtot, post.reshape(N, C, G))


# speedup vs baseline: 2.0970x; 1.0062x over previous
import functools

import jax
import jax.numpy as jnp
from jax import lax
from jax.experimental import pallas as pl
from jax.experimental.pallas import tpu as pltpu
from jax.experimental.pallas import tpu_sc as plsc

N = 10000
E = 160000
C = 16
M = 32
G = 16
CG = C * G

NC = 2
NS = 16
HALF = N // NC
TRASH = HALF
ACC_ROWS = 5120
KC = 64
SUPER = 512
NSUP = 20
EDGES_PER_SUB = NSUP * SUPER
E_PAD = NS * EDGES_PER_SUB
CAP = 768


def _sc_segment_sum(ph2, dst_pad, src_pad):
  mesh = plsc.VectorSubcoreMesh(core_axis_name="c", subcore_axis_name="s")

  @functools.partial(
      pl.kernel,
      out_type=(
          jax.ShapeDtypeStruct((N, CG), jnp.float32),
          jax.ShapeDtypeStruct((N, 16), jnp.float32),
      ),
      mesh=mesh,
      compiler_params=pltpu.CompilerParams(use_tc_tiling_on_sc=False,
                                           needs_layout_passes=False),
      scratch_types=[
          pltpu.VMEM_SHARED((ACC_ROWS, CG), jnp.float32),
          pltpu.VMEM_SHARED((ACC_ROWS, 16), jnp.float32),
          pltpu.VMEM((SUPER,), jnp.int32),
          pltpu.VMEM((SUPER,), jnp.int32),
          pltpu.VMEM((CAP,), jnp.int32),
          pltpu.VMEM((CAP,), jnp.int32),
          pltpu.VMEM((1, KC), jnp.int32),
          pltpu.VMEM((1, KC), jnp.int32),
          pltpu.VMEM((KC, CG), jnp.float32),
          pltpu.VMEM((KC, CG), jnp.float32),
          pltpu.VMEM((KC, 16), jnp.float32),
          pltpu.VMEM((KC, 16), jnp.float32),
          pltpu.SemaphoreType.DMA,
          pltpu.SemaphoreType.DMA,
          pltpu.SemaphoreType.DMA,
          pltpu.SemaphoreType.DMA,
          pltpu.SemaphoreType.DMA,
          pltpu.SemaphoreType.DMA,
          pltpu.SemaphoreType.DMA,
          pltpu.SemaphoreType.DMA,
      ],
  )
  def body(ph_hbm, dst_hbm, src_hbm, sums_hbm, cnts_hbm,
           acc, cacc, dst_st, src_st, ldc, srcc, ldxa, ldxb, rowsa,
           rowsb, ones16, z16, semid, semis, semga, semgb, semsa,
           semsb, semca, semcb):
    cid = lax.axis_index("c")
    sid = lax.axis_index("s")
    base = cid * HALF

    def fill_row(i, _):
      ones16[i, :] = jnp.full((16,), 1.0, jnp.float32)
      z16[i, :] = jnp.zeros((16,), jnp.float32)
      for j in range(CG // 16):
        rowsa[i, pl.ds(j * 16, 16)] = jnp.zeros((16,), jnp.float32)
      return 0
    lax.fori_loop(0, KC, fill_row, 0)

    for k in range(ACC_ROWS // KC // NS):
      q = sid + NS * k
      pltpu.sync_copy(rowsa, acc.at[pl.ds(q * KC, KC)])
      pltpu.sync_copy(z16, cacc.at[pl.ds(q * KC, KC)])
    plsc.subcore_barrier()

    ebase = sid * EDGES_PER_SUB

    def fill_ldx(ldx, t):
      for g in range(KC // 16):
        ldx[0, pl.ds(g * 16, 16)] = ldc[pl.ds(t * KC + g * 16, 16)]

    def process_chunk(t):
      gd = pltpu.async_copy(ph_hbm.at[srcc.at[pl.ds(t * KC, KC)]], rowsa,
                            semga)
      fill_ldx(ldxa, t)
      cd = pltpu.async_copy(ones16, cacc.at[ldxa.at[0]], semca, add=True)
      gd.wait()
      pltpu.sync_copy(rowsa, acc.at[ldxa.at[0]], add=True)
      cd.wait()

    def process_pair(t2, nchunks):
      t0 = 2 * t2
      has_b = (t0 + 1) < nchunks
      prev_b = (t2 > 0) & ((t0 - 1) < nchunks)

      @pl.when(t2 > 0)
      def _():
        pltpu.make_async_copy(rowsa, acc.at[ldxa.at[0]], semsa).wait()
        pltpu.make_async_copy(ones16, cacc.at[ldxa.at[0]], semca).wait()
      ga = pltpu.async_copy(ph_hbm.at[srcc.at[pl.ds(t0 * KC, KC)]],
                            rowsa, semga)
      fill_ldx(ldxa, t0)
      pltpu.async_copy(ones16, cacc.at[ldxa.at[0]], semca, add=True)

      @pl.when(prev_b)
      def _():
        pltpu.make_async_copy(rowsb, acc.at[ldxb.at[0]], semsb).wait()
        pltpu.make_async_copy(ones16, cacc.at[ldxb.at[0]], semcb).wait()

      @pl.when(has_b)
      def _():
        pltpu.async_copy(ph_hbm.at[srcc.at[pl.ds((t0 + 1) * KC, KC)]],
                         rowsb, semgb)
        fill_ldx(ldxb, t0 + 1)
        pltpu.async_copy(ones16, cacc.at[ldxb.at[0]], semcb, add=True)

      ga.wait()
      pltpu.async_copy(rowsa, acc.at[ldxa.at[0]], semsa, add=True)

      @pl.when(has_b)
      def _():
        pltpu.make_async_copy(ph_hbm.at[srcc.at[pl.ds((t0 + 1) * KC, KC)]],
                              rowsb, semgb).wait()
        pltpu.async_copy(rowsb, acc.at[ldxb.at[0]], semsb, add=True)

    pltpu.async_copy(dst_hbm.at[pl.ds(ebase, SUPER)], dst_st, semid)
    pltpu.async_copy(src_hbm.at[pl.ds(ebase, SUPER)], src_st, semis)

    def super_chunk(S, n):
      off = ebase + S * SUPER
      pltpu.make_async_copy(dst_hbm.at[pl.ds(off, SUPER)], dst_st,
                            semid).wait()
      pltpu.make_async_copy(src_hbm.at[pl.ds(off, SUPER)], src_st,
                            semis).wait()
      for g in range(SUPER // 16):
        d = dst_st[pl.ds(g * 16, 16)]
        sv = src_st[pl.ds(g * 16, 16)]
        l = d - base
        ok = (l >= 0) & (l < HALF)
        oki = jnp.where(ok, jnp.full((16,), 1, jnp.int32),
                        jnp.zeros((16,), jnp.int32))
        incl = plsc.cumsum(oki)
        lane = lax.iota(jnp.int32, 16)
        nv = jnp.full((16,), n, jnp.int32)
        pos = jnp.where(ok, nv + incl - 1, CAP - 16 + lane)
        plsc.store_scatter(ldc, [pos], l)
        plsc.store_scatter(srcc, [pos], sv)
        n = n + jnp.max(incl)
      @pl.when(S < NSUP - 1)
      def _():
        off2 = off + SUPER
        pltpu.async_copy(dst_hbm.at[pl.ds(off2, SUPER)], dst_st, semid)
        pltpu.async_copy(src_hbm.at[pl.ds(off2, SUPER)], src_st, semis)

      nchunks = n // KC
      npairs = (nchunks + 1) // 2
      lax.fori_loop(0, npairs,
                    lambda t2, _: (process_pair(t2, nchunks), 0)[1], 0)

      @pl.when(nchunks > 0)
      def _():
        pltpu.make_async_copy(rowsa, acc.at[ldxa.at[0]], semsa).wait()
        pltpu.make_async_copy(ones16, cacc.at[ldxa.at[0]], semca).wait()

      @pl.when((nchunks > 0) & (lax.rem(nchunks, 2) == 0))
      def _():
        pltpu.make_async_copy(rowsb, acc.at[ldxb.at[0]], semsb).wait()
        pltpu.make_async_copy(ones16, cacc.at[ldxb.at[0]], semcb).wait()
      rem_off = nchunks * KC
      for g in range(KC // 16):
        lv = ldc[pl.ds(rem_off + g * 16, 16)]
        sv = srcc[pl.ds(rem_off + g * 16, 16)]
        ldc[pl.ds(g * 16, 16)] = lv
        srcc[pl.ds(g * 16, 16)] = sv
      return n - rem_off

    n_fin = lax.fori_loop(0, NSUP, super_chunk, jnp.int32(0))

    @pl.when(n_fin > 0)
    def _():
      for g in range(KC // 16):
        ldc[pl.ds(n_fin + g * 16, 16)] = jnp.full((16,), TRASH, jnp.int32)
        srcc[pl.ds(n_fin + g * 16, 16)] = jnp.zeros((16,), jnp.int32)
      process_chunk(0)

    plsc.subcore_barrier()

    nq = HALF // 200
    for k in range((nq + NS - 1) // NS):
      q = sid + NS * k
      @pl.when(q < nq)
      def _():
        r0 = q * 200
        pltpu.sync_copy(acc.at[pl.ds(r0, 200)],
                        sums_hbm.at[pl.ds(base + r0, 200)])
        pltpu.sync_copy(cacc.at[pl.ds(r0, 200)],
                        cnts_hbm.at[pl.ds(base + r0, 200)])

  return body(ph2, dst_pad, src_pad)


def _tc_body(sums_ref, cnts_ref, x_ref, lamqt_ref, lamb2_ref,
             logtot_ref, post_ref):
  f32 = jnp.float32
  lamqt = lamqt_ref[...]
  qm = jnp.max(lamqt, axis=1, keepdims=True)
  qe = jnp.exp(lamqt - qm)
  qs = qe / jnp.sum(qe, axis=1, keepdims=True)
  r16 = lax.broadcasted_iota(jnp.int32, (C, CG), 0)
  c256 = lax.broadcasted_iota(jnp.int32, (C, CG), 1)
  e16 = (lax.div(c256, G) == r16).astype(f32)
  qsel = jnp.dot(qs, e16, preferred_element_type=f32)
  rr = lax.broadcasted_iota(jnp.int32, (CG, CG), 0)
  cc = lax.broadcasted_iota(jnp.int32, (CG, CG), 1)
  w = qsel * (lax.rem(rr, G) == lax.rem(cc, G)).astype(f32)

  lamb2 = lamb2_ref[...]
  bm = jnp.max(lamb2, axis=0, keepdims=True)
  be = jnp.exp(lamb2 - bm)
  bs = be / jnp.sum(be, axis=0, keepdims=True)

  nb = sums_ref.shape[0]
  cm = jnp.maximum(cnts_ref[...][:, 0:1], 1.0)
  aggr = sums_ref[...] / cm

  qa = jnp.dot(aggr, w, preferred_element_type=f32)

  xb = x_ref[...]
  mio = lax.broadcasted_iota(jnp.int32, (nb, M), 1)
  oh = (xb == mio).astype(f32)
  bn = jnp.dot(oh, bs, preferred_element_type=f32)

  tmp = bn * qa
  sr = lax.broadcasted_iota(jnp.int32, (CG, G), 0)
  sc = lax.broadcasted_iota(jnp.int32, (CG, G), 1)
  s_mat = (lax.rem(sr, G) == sc).astype(f32)
  total = jnp.dot(tmp, s_mat, preferred_element_type=f32)
  tr = lax.broadcasted_iota(jnp.int32, (G, CG), 0)
  tc = lax.broadcasted_iota(jnp.int32, (G, CG), 1)
  st_mat = (tr == lax.rem(tc, G)).astype(f32)
  totb = jnp.dot(total, st_mat, preferred_element_type=f32)

  logtot_ref[...] = jnp.log(total)
  post_ref[...] = tmp / totb


def _tc_dense(sums, cnts, x2, lamqt, lamb2):
  nb = 1000
  grid = N // nb
  return pl.pallas_call(
      _tc_body,
      grid=(grid,),
      in_specs=[
          pl.BlockSpec((nb, CG), lambda i: (i, 0)),
          pl.BlockSpec((nb, 16), lambda i: (i, 0)),
          pl.BlockSpec((nb, 1), lambda i: (i, 0)),
          pl.BlockSpec((CG, C), lambda i: (0, 0)),
          pl.BlockSpec((M, CG), lambda i: (0, 0)),
      ],
      out_specs=[
          pl.BlockSpec((nb, G), lambda i: (i, 0)),
          pl.BlockSpec((nb, CG), lambda i: (i, 0)),
      ],
      out_shape=[
          jax.ShapeDtypeStruct((N, G), jnp.float32),
          jax.ShapeDtypeStruct((N, CG), jnp.float32),
      ],
  )(sums, cnts, x2, lamqt, lamb2)


def kernel(x, prev_h, edge_index, lambda_Q, lambda_B):
  ph2 = prev_h.reshape(N, CG)
  dst = edge_index[0]
  src = edge_index[1]
  pad = E_PAD - E
  dst_pad = jnp.concatenate([dst, jnp.full((pad,), -1, dst.dtype)])
  src_pad = jnp.concatenate([src, jnp.zeros((pad,), src.dtype)])

  sums, cnts = _sc_segment_sum(ph2, dst_pad.astype(jnp.int32),
                               src_pad.astype(jnp.int32))

  lamqt = jnp.transpose(lambda_Q, (1, 2, 0)).reshape(CG, C)
  lamb2 = jnp.transpose(lambda_B, (1, 0, 2)).reshape(M, CG)
  x2 = x.reshape(N, 1).astype(jnp.int32)

  logtot, post = _tc_dense(sums, cnts, x2, lamqt, lamb2)
  return (logtot, post.reshape(N, C, G))
